# Initial kernel scaffold; baseline (speedup 1.0000x reference)
import jax
import jax.numpy as jnp
from jax.experimental import pallas as pl


def kernel(h0, h1, edge_index, reindex0, reindex1, params):
    raise NotImplementedError("write your pallas kernel here")


# speedup vs baseline: 6.3498x; 6.3498x over previous
import dataclasses
import functools

import jax
import jax.numpy as jnp
from jax import lax
from jax.experimental import pallas as pl
from jax.experimental.pallas import tpu as pltpu
from jax.experimental.pallas import tpu_sc as plsc

N = 50000
E = 800000
NS0 = 40000
NS1 = 30000
IN = 256

CHK = 112
NPAD = 57344
EPAD = 802816
ECHUNKS = EPAD // CHK
NCHUNKS = NPAD // CHK
NCORES = 2
NSUB = 16
ROWS_PER_TILE = NPAD // NSUB

F32 = jnp.float32
I32 = jnp.int32
PREC = lax.Precision.HIGHEST

_mesh = plsc.VectorSubcoreMesh(core_axis_name="c", subcore_axis_name="s")


def _sc_params(layout_passes=True):
    cp = pltpu.CompilerParams(use_tc_tiling_on_sc=False)
    if not layout_passes:
        cp = dataclasses.replace(cp, needs_layout_passes=False)
    return cp


def _wdot(a, b):
    return lax.dot_general(a, b, (((1,), (0,)), ((), ())),
                           preferred_element_type=F32, precision=PREC)


EDGES_PER_WORKER = EPAD // 32
DEG_WIN = 3584
DEG_NW = EDGES_PER_WORKER // DEG_WIN


def _sc_degrees(src_pad, dst_pad):
    @functools.partial(
        pl.kernel,
        out_type=jax.ShapeDtypeStruct((2, 2, 16, 400, 128), F32),
        mesh=_mesh,
        scratch_types=[
            pltpu.VMEM((400, 128), F32),
            pltpu.VMEM((400, 128), F32),
            pltpu.VMEM((DEG_WIN,), I32),
            pltpu.VMEM((DEG_WIN,), I32),
        ],
        compiler_params=_sc_params(layout_passes=False),
    )
    def k(src_ref, dst_ref, out_ref, ho, hi_, wsrc, wdst):
        c = lax.axis_index("c")
        s = lax.axis_index("s")
        wid = c * 16 + s
        base = wid * EDGES_PER_WORKER
        zeros16 = jnp.zeros((16,), F32)
        ones16 = jnp.ones((16,), F32)

        @pl.loop(0, 400)
        def _(j):
            for p in range(8):
                ho[j, pl.ds(p * 16, 16)] = zeros16
                hi_[j, pl.ds(p * 16, 16)] = zeros16

        @pl.loop(0, DEG_NW)
        def _(w):
            wb = base + w * DEG_WIN
            pltpu.sync_copy(src_ref.at[pl.ds(wb, DEG_WIN)], wsrc)
            pltpu.sync_copy(dst_ref.at[pl.ds(wb, DEG_WIN)], wdst)

            @pl.loop(0, DEG_WIN // 16)
            def _(kk):
                i0 = wsrc[pl.ds(kk * 16, 16)]
                plsc.addupdate_scatter(
                    ho, [lax.shift_right_logical(i0, 7),
                         lax.bitwise_and(i0, 127)], ones16)
                i1 = wdst[pl.ds(kk * 16, 16)]
                plsc.addupdate_scatter(
                    hi_, [lax.shift_right_logical(i1, 7),
                          lax.bitwise_and(i1, 127)], ones16)

        pltpu.sync_copy(ho, out_ref.at[0, c, s])
        pltpu.sync_copy(hi_, out_ref.at[1, c, s])

    return k(src_pad, dst_pad)


IMP_WIN = 16


def _sc_impute(t0f, t1f, r0R, r1R):
    out_sds = jax.ShapeDtypeStruct((2, NPAD, 32), F32)

    @functools.partial(
        pl.kernel,
        out_type=[out_sds, out_sds],
        mesh=_mesh,
        scratch_types=[
            pltpu.VMEM((IMP_WIN, CHK), I32),
            pltpu.VMEM((IMP_WIN, CHK), I32),
            pltpu.VMEM((IMP_WIN * CHK, 32), F32),
            pltpu.SemaphoreType.DMA,
        ],
        compiler_params=_sc_params(),
    )
    def k(t0_ref, t1_ref, r0_ref, r1_ref, o0_ref, o1_ref,
          idxw, idx2, rows, sem):
        c = lax.axis_index("c")
        s = lax.axis_index("s")
        off = c * N

        def one(tab_ref, r_ref, o_ref):
            @pl.loop(0, 2)
            def _(w):
                cb = s * 32 + w * IMP_WIN
                pltpu.sync_copy(r_ref.at[pl.ds(cb, IMP_WIN)], idxw)

                @pl.loop(0, IMP_WIN)
                def _(r):
                    for p in range(CHK // 16):
                        sl = pl.ds(p * 16, 16)
                        idx2[r, sl] = idxw[r, sl] + off

                descs = [
                    pltpu.async_copy(tab_ref.at[idx2.at[b]],
                                     rows.at[pl.ds(b * CHK, CHK)], sem)
                    for b in range(IMP_WIN)
                ]
                for d in descs:
                    d.wait()
                nb = cb * CHK
                pltpu.sync_copy(rows, o_ref.at[c, pl.ds(nb, IMP_WIN * CHK)])

        one(t0_ref, r0_ref, o0_ref)
        one(t1_ref, r1_ref, o1_ref)

    return k(t0f, t1f, r0R, r1R)


MP_WIN = 8


def _sc_msgpass(table, srcR, dstR, quarters):
    if quarters == 1:
        tile_chunks = ECHUNKS // (2 * NSUB)
        phases = 1
        nout = 2
    else:
        tile_chunks = ECHUNKS // NSUB
        phases = 2
        nout = 4
    n_win = tile_chunks // MP_WIN
    zrows = 224
    ncopies = ROWS_PER_TILE // zrows

    @functools.partial(
        pl.kernel,
        out_type=jax.ShapeDtypeStruct((nout, NPAD, 16), F32),
        mesh=_mesh,
        scratch_types=[
            pltpu.VMEM((MP_WIN, CHK), I32),
            pltpu.VMEM((MP_WIN, CHK), I32),
            pltpu.VMEM((MP_WIN, CHK), I32),
            pltpu.VMEM((MP_WIN * CHK, 16), F32),
            pltpu.VMEM((zrows, 16), F32),
            pltpu.VMEM_SHARED((NPAD, 16), F32),
            pltpu.SemaphoreType.DMA,
            pltpu.SemaphoreType.DMA,
        ],
        compiler_params=_sc_params(),
    )
    def k(tab_ref, src_ref, dst_ref, out_ref,
          srcw, srcw2, dstw, rows, zbuf, aggS, gsem, ssem):
        c = lax.axis_index("c")
        s = lax.axis_index("s")
        rb = s * ROWS_PER_TILE
        zeros16 = jnp.zeros((16,), F32)

        @pl.loop(0, zrows)
        def _(j):
            zbuf[j, :] = zeros16

        for ph in range(phases):
            @pl.loop(0, ncopies)
            def _(j):
                pltpu.sync_copy(zbuf, aggS.at[pl.ds(rb + j * zrows, zrows)])

            plsc.subcore_barrier()

            if quarters == 1:
                chunk_base = c * (ECHUNKS // 2) + s * tile_chunks
            else:
                chunk_base = s * tile_chunks
                off = (c * 2 + ph) * NPAD

            @pl.loop(0, n_win)
            def _(w):
                cb = chunk_base + w * MP_WIN
                pltpu.sync_copy(src_ref.at[pl.ds(cb, MP_WIN)], srcw)
                pltpu.sync_copy(dst_ref.at[pl.ds(cb, MP_WIN)], dstw)

                if quarters != 1:
                    @pl.loop(0, MP_WIN)
                    def _(r):
                        for p in range(CHK // 16):
                            sl = pl.ds(p * 16, 16)
                            srcw2[r, sl] = srcw[r, sl] + off
                    gather_idx = srcw2
                else:
                    gather_idx = srcw

                descs = [
                    pltpu.async_copy(tab_ref.at[gather_idx.at[b]],
                                     rows.at[pl.ds(b * CHK, CHK)], gsem)
                    for b in range(MP_WIN)
                ]
                for d in descs:
                    d.wait()
                sdescs = [
                    pltpu.async_copy(rows.at[pl.ds(b * CHK, CHK)],
                                     aggS.at[dstw.at[b]], ssem, add=True)
                    for b in range(MP_WIN)
                ]
                for d in sdescs:
                    d.wait()

            plsc.subcore_barrier()
            oq = c if quarters == 1 else c * 2 + ph
            pltpu.sync_copy(aggS.at[pl.ds(rb, ROWS_PER_TILE)],
                            out_ref.at[oq, pl.ds(rb, ROWS_PER_TILE)])
            if ph + 1 < phases:
                plsc.subcore_barrier()

    return k(table, srcR, dstR)


def _tc_stats(x):
    rows = x.shape[0]
    blk = 400
    steps = rows // blk

    def body(x_ref, c_out, s_out, c_acc, s_acc):
        i = pl.program_id(0)

        @pl.when(i == 0)
        def _():
            c_acc[...] = jnp.zeros_like(c_acc)
            s_acc[...] = jnp.zeros_like(s_acc)

        xb = x_ref[...]
        c_acc[...] += lax.dot_general(
            xb, xb, (((0,), (0,)), ((), ())),
            preferred_element_type=F32, precision=PREC)
        s_acc[0:1, :] += jnp.sum(xb, axis=0, keepdims=True)

        @pl.when(i == steps - 1)
        def _():
            c_out[...] = c_acc[...]
            s_out[...] = s_acc[...]

    return pl.pallas_call(
        body,
        grid=(steps,),
        in_specs=[pl.BlockSpec((blk, IN), lambda i: (i, 0))],
        out_specs=[pl.BlockSpec((IN, IN), lambda i: (0, 0)),
                   pl.BlockSpec((8, IN), lambda i: (0, 0))],
        out_shape=[jax.ShapeDtypeStruct((IN, IN), F32),
                   jax.ShapeDtypeStruct((8, IN), F32)],
        scratch_shapes=[pltpu.VMEM((IN, IN), F32), pltpu.VMEM((8, IN), F32)],
    )(x)


def _tc_collapse(C, S, p, n):
    eps = 1e-5

    def body(c_ref, s_ref, w1, b1, g1, be1, w2, b2, g2, be2, wd, bd,
             a3_out, d3_out):
        inv_n = 1.0 / n
        G = c_ref[...] * inv_n
        mu = s_ref[0:1, :] * inv_n
        W1 = w1[...]
        T1 = _wdot(G, W1)
        diag1 = jnp.sum(W1 * T1, axis=0, keepdims=True)
        mean1 = _wdot(mu, W1) + b1[...]
        muw = mean1 - b1[...]
        e2 = diag1 + 2.0 * b1[...] * muw + b1[...] * b1[...]
        var1 = e2 - mean1 * mean1
        a1 = g1[...] * lax.rsqrt(var1 + eps)
        c1 = be1[...] - mean1 * a1

        W2 = w2[...]
        A2 = _wdot(W1 * a1, W2)
        d2 = _wdot(b1[...] * a1 + c1, W2) + b2[...]
        T2 = _wdot(G, A2)
        diag2 = jnp.sum(A2 * T2, axis=0, keepdims=True)
        mean2 = _wdot(mu, A2) + d2
        mua = mean2 - d2
        e2b = diag2 + 2.0 * d2 * mua + d2 * d2
        var2 = e2b - mean2 * mean2
        a2 = g2[...] * lax.rsqrt(var2 + eps)
        c2 = be2[...] - mean2 * a2

        Wd = wd[...]
        a3_out[...] = _wdot(A2 * a2, Wd)
        d3_out[...] = _wdot(d2 * a2 + c2, Wd) + bd[...]

    args = [C, S, p['W1'], p['b1'].reshape(1, -1), p['g1'].reshape(1, -1),
            p['be1'].reshape(1, -1), p['W2'], p['b2'].reshape(1, -1),
            p['g2'].reshape(1, -1), p['be2'].reshape(1, -1), p['Wd'],
            p['bd'].reshape(1, -1)]
    return pl.pallas_call(
        body,
        out_shape=[jax.ShapeDtypeStruct((IN, 64), F32),
                   jax.ShapeDtypeStruct((1, 64), F32)],
    )(*args)


def _tc_dec(x, A3, d3):
    rows = x.shape[0]
    blk = 400
    steps = rows // blk

    def body(x_ref, a_ref, d_ref, o_ref):
        o_ref[...] = _wdot(x_ref[...], a_ref[...]) + d_ref[...]

    return pl.pallas_call(
        body,
        grid=(steps,),
        in_specs=[pl.BlockSpec((blk, IN), lambda i: (i, 0)),
                  pl.BlockSpec((IN, 64), lambda i: (0, 0)),
                  pl.BlockSpec((1, 64), lambda i: (0, 0))],
        out_specs=pl.BlockSpec((blk, 64), lambda i: (i, 0)),
        out_shape=jax.ShapeDtypeStruct((rows, 64), F32),
    )(x, A3, d3)


def _key_xform(u):
    low31 = jnp.int32(0x7FFFFFFF)
    return lax.bitwise_xor(
        u, lax.bitwise_and(lax.shift_right_arithmetic(u, 31), low31))


def _tc_keys(dec, ns):
    half = ns // 2
    blk = 1000
    steps = half // blk

    def body(a_ref, b_ref, o_ref):
        o_ref[:, :64] = _key_xform(lax.bitcast_convert_type(a_ref[...], I32))
        o_ref[:, 64:] = _key_xform(lax.bitcast_convert_type(b_ref[...], I32))

    return pl.pallas_call(
        body,
        grid=(steps,),
        in_specs=[pl.BlockSpec((blk, 64), lambda i: (i, 0)),
                  pl.BlockSpec((blk, 64), lambda i: (i + steps, 0))],
        out_specs=pl.BlockSpec((blk, 128), lambda i: (i, 0)),
        out_shape=jax.ShapeDtypeStruct((half, 128), I32),
    )(dec, dec)


def _tc_median(keys2, ns):
    kth = (ns - 1) // 2

    def body(k_ref, o_ref):
        msb = jnp.int32(-2**31)
        km2 = k_ref[...]

        def step(it, P):
            b = 31 - it
            bit = lax.shift_left(jnp.int32(1), b)
            cand = lax.bitwise_or(P, bit)
            cs = lax.bitwise_xor(cand, msb)
            csw = jnp.concatenate([cs, cs], axis=1)
            cnt128 = jnp.sum(jnp.where(km2 < csw, 1, 0),
                             axis=0, keepdims=True)
            cnt = cnt128[:, :64] + cnt128[:, 64:]
            return jnp.where(cnt <= kth, cand, P)

        P = lax.fori_loop(0, 32, step, jnp.zeros((1, 64), I32))
        kmed = lax.bitwise_xor(P, msb)
        o_ref[...] = lax.bitcast_convert_type(_key_xform(kmed), F32)

    return pl.pallas_call(
        body,
        out_shape=jax.ShapeDtypeStruct((1, 64), F32),
    )(keys2)


def _tc_table(dec, med, ns):
    blk = 1000
    dsteps = ns // blk
    steps = N // blk

    def body(d_ref, m_ref, o_ref):
        i = pl.program_id(0)
        rid = i * blk + lax.broadcasted_iota(I32, (blk, 32), 0)
        d = d_ref[...]
        m = m_ref[...]
        o_ref[0] = jnp.where(rid < ns, d[:, :32],
                             jnp.broadcast_to(m[:, :32], (blk, 32)))
        o_ref[1] = jnp.where(rid < ns, d[:, 32:],
                             jnp.broadcast_to(m[:, 32:], (blk, 32)))

    return pl.pallas_call(
        body,
        grid=(steps,),
        in_specs=[pl.BlockSpec((blk, 64),
                               lambda i: (jnp.minimum(i, dsteps - 1), 0)),
                  pl.BlockSpec((1, 64), lambda i: (0, 0))],
        out_specs=pl.BlockSpec((2, blk, 32), lambda i: (0, i, 0)),
        out_shape=jax.ShapeDtypeStruct((2, N, 32), F32),
    )(dec, med)


def _tc_medtable(dec, ns):
    keys2 = _tc_keys(dec, ns)
    med = _tc_median(keys2, ns)
    return _tc_table(dec, med, ns).reshape(2 * N, 32)


def _tc_norms(degO, degI):
    def body(o_ref, i_ref, out_ref):
        dg = jnp.sum(o_ref[...], axis=0, keepdims=True)
        di = jnp.sum(i_ref[...], axis=0, keepdims=True)
        out_ref[0:1, :] = jnp.where(dg > 0, lax.rsqrt(dg), 0.0)
        out_ref[1:2, :] = jnp.where(di > 0, lax.rsqrt(di), 0.0)

    return pl.pallas_call(
        body,
        out_shape=jax.ShapeDtypeStruct((2, 51200), F32),
    )(degO, degI)


def _tc_scale(imp0, imp1, norm_sd):
    blk = 3584
    steps = NPAD // blk

    def body(i0_ref, i1_ref, n_ref, o_ref):
        ns = n_ref[:, 0:1]
        x0 = (i0_ref[0] + i1_ref[0]) * 0.5 * ns
        x1 = (i0_ref[0 + 1] + i1_ref[0 + 1]) * 0.5 * ns
        o_ref[0] = x0[:, :16]
        o_ref[1] = x0[:, 16:]
        o_ref[2] = x1[:, :16]
        o_ref[3] = x1[:, 16:]

    return pl.pallas_call(
        body,
        grid=(steps,),
        in_specs=[pl.BlockSpec((2, blk, 32), lambda i: (0, i, 0)),
                  pl.BlockSpec((2, blk, 32), lambda i: (0, i, 0)),
                  pl.BlockSpec((blk, 2), lambda i: (i, 0))],
        out_specs=pl.BlockSpec((4, blk, 16), lambda i: (0, i, 0)),
        out_shape=jax.ShapeDtypeStruct((4, NPAD, 16), F32),
    )(imp0, imp1, norm_sd)


def _tc_mid(agg, norm_sd, W0, b0, W1):
    blk = 3584
    steps = NPAD // blk

    def body(a_ref, n_ref, w0_ref, b0_ref, w1_ref, o_ref):
        nd = n_ref[:, 1:2]
        ns = n_ref[:, 0:1]
        a64 = jnp.concatenate([a_ref[0], a_ref[1], a_ref[2], a_ref[3]],
                              axis=1)
        h = jnp.maximum(_wdot(a64 * nd, w0_ref[...]) + b0_ref[...], 0.0)
        o_ref[...] = _wdot(h, w1_ref[...]) * ns

    return pl.pallas_call(
        body,
        grid=(steps,),
        in_specs=[pl.BlockSpec((4, blk, 16), lambda i: (0, i, 0)),
                  pl.BlockSpec((blk, 2), lambda i: (i, 0)),
                  pl.BlockSpec((64, 64), lambda i: (0, 0)),
                  pl.BlockSpec((1, 64), lambda i: (0, 0)),
                  pl.BlockSpec((64, 16), lambda i: (0, 0))],
        out_specs=pl.BlockSpec((blk, 16), lambda i: (i, 0)),
        out_shape=jax.ShapeDtypeStruct((NPAD, 16), F32),
    )(agg, norm_sd, W0, b0.reshape(1, -1), W1)


def _tc_final(p01, norm_sd, b1):
    blk = 1000
    steps = N // blk

    def body(p_ref, n_ref, b_ref, o_ref):
        nd = n_ref[:, 1:2]
        o_ref[...] = (p_ref[0] + p_ref[1]) * nd + b_ref[...]

    return pl.pallas_call(
        body,
        grid=(steps,),
        in_specs=[pl.BlockSpec((2, blk, 16), lambda i: (0, i, 0)),
                  pl.BlockSpec((blk, 2), lambda i: (i, 0)),
                  pl.BlockSpec((1, 16), lambda i: (0, 0))],
        out_specs=pl.BlockSpec((blk, 16), lambda i: (i, 0)),
        out_shape=jax.ShapeDtypeStruct((N, 16), F32),
    )(p01, norm_sd, b1.reshape(1, -1))


def kernel(h0, h1, edge_index, reindex0, reindex1, params):
    src = edge_index[0].astype(I32)
    dst = edge_index[1].astype(I32)
    epad = jnp.full((EPAD - E,), N, I32)
    src_pad = jnp.concatenate([src, epad])
    dst_pad = jnp.concatenate([dst, epad])
    srcR = src_pad.reshape(ECHUNKS, CHK)
    dstR = dst_pad.reshape(ECHUNKS, CHK)
    npad0 = jnp.zeros((NPAD - N,), I32)
    r0R = jnp.concatenate([reindex0.astype(I32), npad0]).reshape(NCHUNKS, CHK)
    r1R = jnp.concatenate([reindex1.astype(I32), npad0]).reshape(NCHUNKS, CHK)

    degp = _sc_degrees(src_pad, dst_pad)
    degf = degp.reshape(2, 32, 51200)
    norm2 = _tc_norms(degf[0], degf[1])
    norm_sd = jnp.concatenate(
        [norm2[:, :N], jnp.zeros((2, NPAD - N), F32)], axis=1).T

    C0, S0 = _tc_stats(h0)
    A3_0, d3_0 = _tc_collapse(C0, S0, params['enc0'], float(NS0))
    dec0 = _tc_dec(h0, A3_0, d3_0)
    t0f = _tc_medtable(dec0, NS0)

    C1, S1 = _tc_stats(h1)
    A3_1, d3_1 = _tc_collapse(C1, S1, params['enc1'], float(NS1))
    dec1 = _tc_dec(h1, A3_1, d3_1)
    t1f = _tc_medtable(dec1, NS1)

    imp0, imp1 = _sc_impute(t0f, t1f, r0R, r1R)
    xsf = _tc_scale(imp0, imp1, norm_sd).reshape(4 * NPAD, 16)

    agg = _sc_msgpass(xsf, srcR, dstR, quarters=4)
    g = _tc_mid(agg, norm_sd, params['gcn0_W'], params['gcn0_b'],
                params['gcn1_W'])

    p01 = _sc_msgpass(g, srcR, dstR, quarters=1)
    return _tc_final(p01, norm_sd, params['gcn1_b'])
